# baseline (speedup 1.0000x reference)
import functools

import jax
import jax.numpy as jnp
from jax import lax
from jax.experimental import pallas as pl
from jax.experimental.pallas import tpu as pltpu
from jax.experimental.pallas import tpu_sc as plsc

_D = 64
_B = 16384
_L = 16
_NC = 2
_NS = 16
_NW = _NC * _NS
_BPW = _B // _NW
_C = 128
_NCHUNK = _BPW // _C
_G = _D // _L
_P = 2 * _D

_mesh = plsc.VectorSubcoreMesh(core_axis_name="c", subcore_axis_name="s")
_params = pltpu.CompilerParams(
    needs_layout_passes=False, use_tc_tiling_on_sc=True)


def _stage_indices(srcs, base, idx_vs, idx_sh, idx_s, sid):
    for k, (src, idx_v) in enumerate(zip(srcs, idx_vs)):
        pltpu.sync_copy(src.at[pl.ds(base, _BPW)], idx_v)
        pltpu.sync_copy(idx_v, idx_sh.at[sid, pl.ds(k * _BPW, _BPW)])
    pltpu.sync_copy(idx_sh.at[sid], idx_s)


@functools.partial(
    pl.kernel,
    mesh=_mesh,
    compiler_params=_params,
    out_type=[
        jax.ShapeDtypeStruct((_B * _P,), jnp.float32),
        jax.ShapeDtypeStruct((_NW, _L), jnp.float32),
    ],
    scratch_types=[
        pltpu.SMEM((2 * _BPW,), jnp.int32),
        pltpu.VMEM((_BPW,), jnp.int32),
        pltpu.VMEM((_BPW,), jnp.int32),
        pltpu.VMEM_SHARED((_NS, 2 * _BPW), jnp.int32),
        pltpu.VMEM((_C, _D), jnp.float32),
        pltpu.VMEM((_C, _D), jnp.float32),
        pltpu.VMEM((_C, _D), jnp.float32),
        pltpu.VMEM((_C, _D), jnp.float32),
        pltpu.VMEM((_C * _P,), jnp.float32),
        pltpu.VMEM((_L,), jnp.float32),
        pltpu.SemaphoreType.DMA,
    ],
)
def _sc_ent(heads, tails, ent_re, ent_im,
            p_out, sq_out,
            idx_s, idx_vh, idx_vt, idx_sh,
            bhre, bhim, btre, btim, p_buf, sq_buf, sem):
    sid = lax.axis_index("s")
    wid = sid * _NC + lax.axis_index("c")
    base = wid * _BPW

    _stage_indices((heads, tails), base, (idx_vh, idx_vt), idx_sh, idx_s, sid)

    def chunk_body(ci, sq_carry):
        off = base + ci * _C
        loc = ci * _C

        def issue_body(i, _):
            h = idx_s[loc + i]
            t = idx_s[_BPW + loc + i]
            pltpu.async_copy(ent_re.at[h], bhre.at[i], sem)
            pltpu.async_copy(ent_im.at[h], bhim.at[i], sem)
            pltpu.async_copy(ent_re.at[t], btre.at[i], sem)
            pltpu.async_copy(ent_im.at[t], btim.at[i], sem)
            return 0

        lax.fori_loop(0, _C, issue_body, 0)
        for buf in (bhre, bhim, btre, btim):
            pltpu.make_async_copy(ent_re.at[pl.ds(0, _C)], buf, sem).wait()

        def tri_body(i, carry):
            sq1, sq2 = carry
            for g in range(_G):
                sl = pl.ds(g * _L, _L)
                hre = bhre[i, sl]
                him = bhim[i, sl]
                tre = btre[i, sl]
                tim = btim[i, sl]
                p_buf[pl.ds(i * _P + g * _L, _L)] = hre * tre + him * tim
                p_buf[pl.ds(i * _P + _D + g * _L, _L)] = hre * tim - him * tre
                sq1 = sq1 + (hre * hre + him * him)
                sq2 = sq2 + (tre * tre + tim * tim)
            return (sq1, sq2)

        sq_carry = lax.fori_loop(0, _C, tri_body, sq_carry)
        pltpu.sync_copy(p_buf, p_out.at[pl.ds(off * _P, _C * _P)])
        return sq_carry

    zero = jnp.zeros((_L,), jnp.float32)
    sq1, sq2 = lax.fori_loop(0, _NCHUNK, chunk_body, (zero, zero))
    sq_buf[...] = sq1 + sq2
    pltpu.sync_copy(sq_buf, sq_out.at[wid])


@functools.partial(
    pl.kernel,
    mesh=_mesh,
    compiler_params=_params,
    out_type=[
        jax.ShapeDtypeStruct((_B * _L,), jnp.float32),
        jax.ShapeDtypeStruct((_NW, _L), jnp.float32),
    ],
    scratch_types=[
        pltpu.SMEM((_BPW,), jnp.int32),
        pltpu.VMEM((_BPW,), jnp.int32),
        pltpu.VMEM_SHARED((_NS, _BPW), jnp.int32),
        pltpu.VMEM((_C, _D), jnp.float32),
        pltpu.VMEM((_C, _D), jnp.float32),
        pltpu.VMEM((_C * _P,), jnp.float32),
        pltpu.VMEM((_C * _L,), jnp.float32),
        pltpu.VMEM((_L,), jnp.float32),
        pltpu.SemaphoreType.DMA,
    ],
)
def _sc_rel(rels, rel_re, rel_im, p_in,
            score_out, sq_out,
            idx_s, idx_vr, idx_sh,
            brre, brim, p_buf, score_buf, sq_buf, sem):
    sid = lax.axis_index("s")
    wid = sid * _NC + lax.axis_index("c")
    base = wid * _BPW

    _stage_indices((rels,), base, (idx_vr,), idx_sh, idx_s, sid)

    def chunk_body(ci, sq_carry):
        off = base + ci * _C
        loc = ci * _C

        cp_p = pltpu.async_copy(
            p_in.at[pl.ds(off * _P, _C * _P)], p_buf, sem)

        def issue_body(i, _):
            r = idx_s[loc + i]
            pltpu.async_copy(rel_re.at[r], brre.at[i], sem)
            pltpu.async_copy(rel_im.at[r], brim.at[i], sem)
            return 0

        lax.fori_loop(0, _C, issue_body, 0)
        for buf in (brre, brim):
            pltpu.make_async_copy(rel_re.at[pl.ds(0, _C)], buf, sem).wait()
        cp_p.wait()

        def tri_body(i, carry):
            sq3 = carry
            score16 = jnp.zeros((_L,), jnp.float32)
            for g in range(_G):
                sl = pl.ds(g * _L, _L)
                rre = brre[i, sl]
                rim = brim[i, sl]
                p1 = p_buf[pl.ds(i * _P + g * _L, _L)]
                p2 = p_buf[pl.ds(i * _P + _D + g * _L, _L)]
                score16 = score16 + rre * p1 + rim * p2
                sq3 = sq3 + (rre * rre + rim * rim)
            score_buf[pl.ds(i * _L, _L)] = score16
            return sq3

        sq_carry = lax.fori_loop(0, _C, tri_body, sq_carry)
        pltpu.sync_copy(score_buf, score_out.at[pl.ds(off * _L, _C * _L)])
        return sq_carry

    sq3 = lax.fori_loop(0, _NCHUNK, chunk_body, jnp.zeros((_L,), jnp.float32))
    sq_buf[...] = sq3
    pltpu.sync_copy(sq_buf, sq_out.at[wid])


def _tc_finish_body(part_ref, labels_ref, sqe_ref, sqr_ref, out_ref):
    part = part_ref[...]
    row = lax.broadcasted_iota(jnp.int32, (128, 8), 0)
    col = lax.broadcasted_iota(jnp.int32, (128, 8), 1)
    sel = (row // _L == col).astype(jnp.float32)
    score8 = -jax.lax.dot_general(
        part, sel, (((1,), (0,)), ((), ())),
        preferred_element_type=jnp.float32)
    x = score8 * labels_ref[...]
    sp = jnp.maximum(x, 0.0) + jnp.log(1.0 + jnp.exp(-jnp.abs(x)))
    regul = (jnp.sum(sqe_ref[...]) + jnp.sum(sqr_ref[...])) * (
        0.01 / (_B * _D))
    total = jnp.sum(sp) * (1.0 / _B) + regul
    out_ref[...] = jnp.broadcast_to(total, (1, 1))


def _tc_finish(part, labels8, sqe, sqr):
    return pl.pallas_call(
        _tc_finish_body,
        out_shape=jax.ShapeDtypeStruct((1, 1), jnp.float32),
    )(part, labels8, sqe, sqr)


def kernel(batch, labels, ent_re, ent_im, rel_re, rel_im):
    heads = batch[:, 0]
    rels = batch[:, 1]
    tails = batch[:, 2]
    p, sqe = _sc_ent(heads, tails, ent_re, ent_im)
    part, sqr = _sc_rel(rels, rel_re, rel_im, p)
    loss = _tc_finish(
        part.reshape(_B // 8, 128), labels.reshape(_B // 8, 8), sqe, sqr)
    return loss[0, 0]
